# Initial kernel scaffold; baseline (speedup 1.0000x reference)
import jax
import jax.numpy as jnp
from jax.experimental import pallas as pl


def kernel(target):
    raise NotImplementedError("write your pallas kernel here")


# speedup vs baseline: 2.8679x; 2.8679x over previous
import jax
import jax.numpy as jnp
from jax import lax
from jax.experimental import pallas as pl

NUM_K = 150
C, H, W = 4, 512, 512
ROWS = 16


def _body(t_ref, o_ref):
    label = t_ref[0].astype(jnp.int32)
    kk = lax.broadcasted_iota(jnp.int32, (NUM_K, ROWS, W), 0)
    o_ref[0:NUM_K] = (kk == label[None]).astype(jnp.float32)
    o_ref[NUM_K : NUM_K + C - 1] = t_ref[1:C]


def kernel(target):
    grid = (H // ROWS,)
    return pl.pallas_call(
        _body,
        grid=grid,
        in_specs=[pl.BlockSpec((C, ROWS, W), lambda i: (0, i, 0))],
        out_specs=pl.BlockSpec((NUM_K + C - 1, ROWS, W), lambda i: (0, i, 0)),
        out_shape=jax.ShapeDtypeStruct((NUM_K + C - 1, H, W), jnp.float32),
    )(target)
